# Initial kernel scaffold; baseline (speedup 1.0000x reference)
import functools

import jax
import jax.numpy as jnp
from jax import lax
from jax.experimental import pallas as pl
from jax.experimental.pallas import tpu as pltpu
from jax.experimental.pallas import tpu_sc as plsc

N_NODES = 10000
N_EDGES = 320000
F = 128
EXTRA = 16

NC = 2
NS = 16
L = 16

NW = NC * NS
CH = 128
NP = 10240
ROWS_PER_TILE = NP // NS
EPT = 10112
EP = EPT * NW
NCHUNK = EPT // CH
DL = 16

_SELU_ALPHA = 1.6732632423543772
_SELU_SCALE = 1.0507009873554805


def _selu(x):
    return _SELU_SCALE * jnp.where(x > 0, x, _SELU_ALPHA * (jnp.exp(x) - 1.0))


def _mesh():
    return plsc.VectorSubcoreMesh(core_axis_name="c", subcore_axis_name="s")


def _fill_2d(ref, rows, value):
    cols = ref.shape[1]

    @pl.loop(0, rows)
    def _(i):
        @pl.loop(0, cols, step=L)
        def _(j):
            ref[i, pl.ds(j, L)] = jnp.full((L,), value, dtype=ref.dtype)


def _degrees_body(src_hbm, dst_hbm, dego_hbm, degi_hbm,
                  idx_v, ones_v, zeros_v, acc_o, acc_i):
    cid = lax.axis_index("c")
    sid = lax.axis_index("s")
    wid = cid * NS + sid

    _fill_2d(ones_v, CH, 1.0)
    _fill_2d(zeros_v, CH, 0.0)

    @pl.loop(0, ROWS_PER_TILE, step=CH)
    def _(r):
        pltpu.sync_copy(zeros_v, acc_o.at[pl.ds(sid * ROWS_PER_TILE + r, CH)])
        pltpu.sync_copy(zeros_v, acc_i.at[pl.ds(sid * ROWS_PER_TILE + r, CH)])

    plsc.subcore_barrier()

    @pl.loop(0, NCHUNK)
    def _(k):
        base = wid * EPT + k * CH
        pltpu.sync_copy(src_hbm.at[pl.ds(base, CH)], idx_v)
        pltpu.sync_copy(ones_v, acc_o.at[idx_v], add=True)
        pltpu.sync_copy(dst_hbm.at[pl.ds(base, CH)], idx_v)
        pltpu.sync_copy(ones_v, acc_i.at[idx_v], add=True)

    plsc.subcore_barrier()

    r0 = sid * ROWS_PER_TILE
    pltpu.sync_copy(acc_o.at[pl.ds(r0, ROWS_PER_TILE)],
                    dego_hbm.at[cid].at[pl.ds(r0, ROWS_PER_TILE)])
    pltpu.sync_copy(acc_i.at[pl.ds(r0, ROWS_PER_TILE)],
                    degi_hbm.at[cid].at[pl.ds(r0, ROWS_PER_TILE)])


def _sc_degrees(src, dst):
    k = pl.kernel(
        _degrees_body,
        out_type=(
            jax.ShapeDtypeStruct((NC, NP, DL), jnp.float32),
            jax.ShapeDtypeStruct((NC, NP, DL), jnp.float32),
        ),
        mesh=_mesh(),
        scratch_types=[
            pltpu.VMEM((CH,), jnp.int32),
            pltpu.VMEM((CH, DL), jnp.float32),
            pltpu.VMEM((CH, DL), jnp.float32),
            pltpu.VMEM_SHARED((NP, DL), jnp.float32),
            pltpu.VMEM_SHARED((NP, DL), jnp.float32),
        ],
    )
    return k(src, dst)


def _agg_body(h_hbm, src_hbm, dst_hbm, out_hbm,
              idx_s, idx_d, rows_v, zeros_v, acc):
    cid = lax.axis_index("c")
    sid = lax.axis_index("s")
    wid = cid * NS + sid

    _fill_2d(zeros_v, CH, 0.0)

    @pl.loop(0, ROWS_PER_TILE, step=CH)
    def _(r):
        pltpu.sync_copy(zeros_v, acc.at[pl.ds(sid * ROWS_PER_TILE + r, CH)])

    plsc.subcore_barrier()

    @pl.loop(0, NCHUNK)
    def _(k):
        base = wid * EPT + k * CH
        pltpu.sync_copy(src_hbm.at[pl.ds(base, CH)], idx_s)
        pltpu.sync_copy(h_hbm.at[idx_s], rows_v)
        pltpu.sync_copy(dst_hbm.at[pl.ds(base, CH)], idx_d)
        pltpu.sync_copy(rows_v, acc.at[idx_d], add=True)

    plsc.subcore_barrier()

    r0 = sid * ROWS_PER_TILE
    pltpu.sync_copy(acc.at[pl.ds(r0, ROWS_PER_TILE)],
                    out_hbm.at[cid].at[pl.ds(r0, ROWS_PER_TILE)])


def _sc_aggregate(h, src, dst):
    k = pl.kernel(
        _agg_body,
        out_type=jax.ShapeDtypeStruct((NC, NP, F), jnp.float32),
        mesh=_mesh(),
        scratch_types=[
            pltpu.VMEM((CH,), jnp.int32),
            pltpu.VMEM((CH,), jnp.int32),
            pltpu.VMEM((CH, F), jnp.float32),
            pltpu.VMEM((CH, F), jnp.float32),
            pltpu.VMEM_SHARED((NP, F), jnp.float32),
        ],
    )
    return k(h, src, dst)


def _norm_body(dego_ref, degi_ref, x_ref, hs_ref, ns_ref, nd_ref):
    dego = dego_ref[0] + dego_ref[1]
    degi = degi_ref[0] + degi_ref[1]
    do = dego[:, 0:1]
    di = degi[:, 0:1]
    ns = jnp.where(do > 0, lax.rsqrt(do), 0.0)
    nd = jnp.where(di > 0, lax.rsqrt(di), 0.0)
    ns_ref[...] = ns
    nd_ref[...] = nd
    hs_ref[...] = x_ref[...] * ns


def _tc_norms(dego, degi, xp):
    return pl.pallas_call(
        _norm_body,
        out_shape=(
            jax.ShapeDtypeStruct((NP, F), jnp.float32),
            jax.ShapeDtypeStruct((NP, 1), jnp.float32),
            jax.ShapeDtypeStruct((NP, 1), jnp.float32),
        ),
    )(dego, degi, xp)


RB = 1024


def _layer_body(m_ref, nd_ref, ns_ref, w_ref, b_ref, out_ref):
    a = (m_ref[0] + m_ref[1]) * nd_ref[...]
    h = lax.dot_general(a, w_ref[...], (((1,), (0,)), ((), ())),
                        precision=lax.Precision.HIGHEST,
                        preferred_element_type=jnp.float32)
    h = _selu(h + b_ref[...])
    out_ref[...] = h * ns_ref[...]


def _tc_layer(m_p, nd, ns, w, b):
    grid = NP // RB
    return pl.pallas_call(
        _layer_body,
        grid=(grid,),
        in_specs=[
            pl.BlockSpec((NC, RB, F), lambda i: (0, i, 0)),
            pl.BlockSpec((RB, 1), lambda i: (i, 0)),
            pl.BlockSpec((RB, 1), lambda i: (i, 0)),
            pl.BlockSpec((F, F), lambda i: (0, 0)),
            pl.BlockSpec((1, F), lambda i: (0, 0)),
        ],
        out_specs=pl.BlockSpec((RB, F), lambda i: (i, 0)),
        out_shape=jax.ShapeDtypeStruct((NP, F), jnp.float32),
    )(m_p, nd, ns, w, b)


def _pool_body(m_ref, nd_ref, w_ref, b_ref, out_ref):
    i = pl.program_id(0)
    a = (m_ref[0] + m_ref[1]) * nd_ref[...]
    h = lax.dot_general(a, w_ref[...], (((1,), (0,)), ((), ())),
                        precision=lax.Precision.HIGHEST,
                        preferred_element_type=jnp.float32)
    h = _selu(h + b_ref[...])
    row = i * RB + lax.broadcasted_iota(jnp.int32, (RB, F), 0)
    h = jnp.where(row < N_NODES, h, 0.0)
    ps = jnp.sum(h, axis=0, keepdims=True)

    @pl.when(i == 0)
    def _():
        out_ref[...] = jnp.zeros_like(out_ref)

    out_ref[...] += ps


def _tc_layer3_pool(m_p, nd, w, b):
    grid = NP // RB
    return pl.pallas_call(
        _pool_body,
        grid=(grid,),
        in_specs=[
            pl.BlockSpec((NC, RB, F), lambda i: (0, i, 0)),
            pl.BlockSpec((RB, 1), lambda i: (i, 0)),
            pl.BlockSpec((F, F), lambda i: (0, 0)),
            pl.BlockSpec((1, F), lambda i: (0, 0)),
        ],
        out_specs=pl.BlockSpec((1, F), lambda i: (0, 0)),
        out_shape=jax.ShapeDtypeStruct((1, F), jnp.float32),
    )(m_p, nd, w, b)


def _head_body(es_ref, fg_ref, w1a_ref, w1b_ref, b1_ref, w2_ref, b2_ref,
               w3_ref, b3_ref, out_ref):
    emb = es_ref[...] * (1.0 / N_NODES)
    t = (lax.dot_general(emb, w1a_ref[...], (((1,), (0,)), ((), ())),
                         precision=lax.Precision.HIGHEST,
                         preferred_element_type=jnp.float32)
         + lax.dot_general(fg_ref[...], w1b_ref[...], (((1,), (0,)), ((), ())),
                           precision=lax.Precision.HIGHEST,
                           preferred_element_type=jnp.float32))
    z = _selu(t + b1_ref[...])
    z = _selu(lax.dot_general(z, w2_ref[...], (((1,), (0,)), ((), ())),
                              precision=lax.Precision.HIGHEST,
                              preferred_element_type=jnp.float32)
              + b2_ref[...])
    out_ref[...] = (lax.dot_general(z, w3_ref[...], (((1,), (0,)), ((), ())),
                                    precision=lax.Precision.HIGHEST,
                                    preferred_element_type=jnp.float32)
                    + b3_ref[...])


def _tc_head(emb_sum, fg, w1a, w1b, b1, w2, b2, w3, b3):
    return pl.pallas_call(
        _head_body,
        out_shape=jax.ShapeDtypeStruct((1, 1), jnp.float32),
    )(emb_sum, fg, w1a, w1b, b1, w2, b2, w3, b3)


def kernel(feats_node, edge_index, feats_graph, W1, b1, W2, b2, W3, b3,
           Wm1, bm1, Wm2, bm2, Wm3, bm3):
    ei = edge_index.astype(jnp.int32)
    pad = jnp.full((EP - N_EDGES,), NP - 1, dtype=jnp.int32)
    src = jnp.concatenate([ei[0], pad])
    dst = jnp.concatenate([ei[1], pad])

    xp = jnp.zeros((NP, F), jnp.float32).at[:N_NODES].set(feats_node)

    dego, degi = _sc_degrees(src, dst)
    hs, ns, nd = _tc_norms(dego, degi, xp)

    m1 = _sc_aggregate(hs, src, dst)
    hs2 = _tc_layer(m1, nd, ns, W1, b1.reshape(1, F))

    m2 = _sc_aggregate(hs2, src, dst)
    hs3 = _tc_layer(m2, nd, ns, W2, b2.reshape(1, F))

    m3 = _sc_aggregate(hs3, src, dst)
    emb_sum = _tc_layer3_pool(m3, nd, W3, b3.reshape(1, F))

    return _tc_head(emb_sum, feats_graph,
                    Wm1[:F], Wm1[F:], bm1.reshape(1, 2 * F),
                    Wm2, bm2.reshape(1, F),
                    Wm3, bm3.reshape(1, 1))


# speedup vs baseline: 3.5262x; 3.5262x over previous
import functools

import jax
import jax.numpy as jnp
from jax import lax
from jax.experimental import pallas as pl
from jax.experimental.pallas import tpu as pltpu
from jax.experimental.pallas import tpu_sc as plsc

N_NODES = 10000
N_EDGES = 320000
F = 128
EXTRA = 16

NC = 2
NS = 16
L = 16

NW = NC * NS
CH = 128
NP = 10240
ROWS_PER_TILE = NP // NS
EPT = 10112
EP = EPT * NW
NCHUNK = EPT // CH
DL = 16

_SELU_ALPHA = 1.6732632423543772
_SELU_SCALE = 1.0507009873554805


def _selu(x):
    return _SELU_SCALE * jnp.where(x > 0, x, _SELU_ALPHA * (jnp.exp(x) - 1.0))


def _mesh():
    return plsc.VectorSubcoreMesh(core_axis_name="c", subcore_axis_name="s")


def _fill_2d(ref, rows, value):
    cols = ref.shape[1]

    @pl.loop(0, rows)
    def _(i):
        @pl.loop(0, cols, step=L)
        def _(j):
            ref[i, pl.ds(j, L)] = jnp.full((L,), value, dtype=ref.dtype)


def _degrees_body(src_hbm, dst_hbm, dego_hbm, degi_hbm,
                  idx_v, acc_o, acc_i):
    cid = lax.axis_index("c")
    sid = lax.axis_index("s")
    wid = cid * NS + sid

    @pl.loop(0, NP, step=L)
    def _(i):
        z = jnp.zeros((L,), jnp.float32)
        acc_o[pl.ds(i, L)] = z
        acc_i[pl.ds(i, L)] = z

    ones = jnp.full((L,), 1.0, jnp.float32)

    @pl.loop(0, NCHUNK)
    def _(k):
        base = wid * EPT + k * CH
        pltpu.sync_copy(src_hbm.at[pl.ds(base, CH)], idx_v)

        @pl.loop(0, CH, step=L)
        def _(j):
            plsc.addupdate_scatter(acc_o, [idx_v[pl.ds(j, L)]], ones)

        pltpu.sync_copy(dst_hbm.at[pl.ds(base, CH)], idx_v)

        @pl.loop(0, CH, step=L)
        def _(j):
            plsc.addupdate_scatter(acc_i, [idx_v[pl.ds(j, L)]], ones)

    pltpu.sync_copy(acc_o, dego_hbm.at[wid])
    pltpu.sync_copy(acc_i, degi_hbm.at[wid])


def _deg_compiler_params():
    import dataclasses
    cp = pltpu.CompilerParams()
    if "needs_layout_passes" in pltpu.CompilerParams.__dataclass_fields__:
        cp = dataclasses.replace(cp, needs_layout_passes=False)
    return cp


def _sc_degrees(src, dst):
    k = pl.kernel(
        _degrees_body,
        out_type=(
            jax.ShapeDtypeStruct((NW, NP), jnp.float32),
            jax.ShapeDtypeStruct((NW, NP), jnp.float32),
        ),
        mesh=_mesh(),
        scratch_types=[
            pltpu.VMEM((CH,), jnp.int32),
            pltpu.VMEM((NP,), jnp.float32),
            pltpu.VMEM((NP,), jnp.float32),
        ],
        compiler_params=_deg_compiler_params(),
    )
    return k(src, dst)


def _agg_body(h_hbm, src_hbm, dst_hbm, out_hbm,
              idx_s, idx_d, rows_v, zeros_v, acc):
    cid = lax.axis_index("c")
    sid = lax.axis_index("s")
    wid = cid * NS + sid

    _fill_2d(zeros_v, CH, 0.0)

    @pl.loop(0, ROWS_PER_TILE, step=CH)
    def _(r):
        pltpu.sync_copy(zeros_v, acc.at[pl.ds(sid * ROWS_PER_TILE + r, CH)])

    plsc.subcore_barrier()

    @pl.loop(0, NCHUNK)
    def _(k):
        base = wid * EPT + k * CH
        pltpu.sync_copy(src_hbm.at[pl.ds(base, CH)], idx_s)
        pltpu.sync_copy(h_hbm.at[idx_s], rows_v)
        pltpu.sync_copy(dst_hbm.at[pl.ds(base, CH)], idx_d)
        pltpu.sync_copy(rows_v, acc.at[idx_d], add=True)

    plsc.subcore_barrier()

    r0 = sid * ROWS_PER_TILE
    pltpu.sync_copy(acc.at[pl.ds(r0, ROWS_PER_TILE)],
                    out_hbm.at[cid].at[pl.ds(r0, ROWS_PER_TILE)])


def _sc_aggregate(h, src, dst):
    k = pl.kernel(
        _agg_body,
        out_type=jax.ShapeDtypeStruct((NC, NP, F), jnp.float32),
        mesh=_mesh(),
        scratch_types=[
            pltpu.VMEM((CH,), jnp.int32),
            pltpu.VMEM((CH,), jnp.int32),
            pltpu.VMEM((CH, F), jnp.float32),
            pltpu.VMEM((CH, F), jnp.float32),
            pltpu.VMEM_SHARED((NP, F), jnp.float32),
        ],
    )
    return k(h, src, dst)


def _norm_body(dego_ref, degi_ref, x_ref, hs_ref, ns_ref, nd_ref):
    do = jnp.sum(dego_ref[...], axis=1, keepdims=True)
    di = jnp.sum(degi_ref[...], axis=1, keepdims=True)
    ns = jnp.where(do > 0, lax.rsqrt(do), 0.0)
    nd = jnp.where(di > 0, lax.rsqrt(di), 0.0)
    ns_ref[...] = ns
    nd_ref[...] = nd
    hs_ref[...] = x_ref[...] * ns


def _tc_norms(dego, degi, xp):
    return pl.pallas_call(
        _norm_body,
        out_shape=(
            jax.ShapeDtypeStruct((NP, F), jnp.float32),
            jax.ShapeDtypeStruct((NP, 1), jnp.float32),
            jax.ShapeDtypeStruct((NP, 1), jnp.float32),
        ),
    )(dego, degi, xp)


RB = 1024


def _layer_body(m_ref, nd_ref, ns_ref, w_ref, b_ref, out_ref):
    a = (m_ref[0] + m_ref[1]) * nd_ref[...]
    h = lax.dot_general(a, w_ref[...], (((1,), (0,)), ((), ())),
                        precision=lax.Precision.HIGHEST,
                        preferred_element_type=jnp.float32)
    h = _selu(h + b_ref[...])
    out_ref[...] = h * ns_ref[...]


def _tc_layer(m_p, nd, ns, w, b):
    grid = NP // RB
    return pl.pallas_call(
        _layer_body,
        grid=(grid,),
        in_specs=[
            pl.BlockSpec((NC, RB, F), lambda i: (0, i, 0)),
            pl.BlockSpec((RB, 1), lambda i: (i, 0)),
            pl.BlockSpec((RB, 1), lambda i: (i, 0)),
            pl.BlockSpec((F, F), lambda i: (0, 0)),
            pl.BlockSpec((1, F), lambda i: (0, 0)),
        ],
        out_specs=pl.BlockSpec((RB, F), lambda i: (i, 0)),
        out_shape=jax.ShapeDtypeStruct((NP, F), jnp.float32),
    )(m_p, nd, ns, w, b)


def _pool_body(m_ref, nd_ref, w_ref, b_ref, out_ref):
    i = pl.program_id(0)
    a = (m_ref[0] + m_ref[1]) * nd_ref[...]
    h = lax.dot_general(a, w_ref[...], (((1,), (0,)), ((), ())),
                        precision=lax.Precision.HIGHEST,
                        preferred_element_type=jnp.float32)
    h = _selu(h + b_ref[...])
    row = i * RB + lax.broadcasted_iota(jnp.int32, (RB, F), 0)
    h = jnp.where(row < N_NODES, h, 0.0)
    ps = jnp.sum(h, axis=0, keepdims=True)

    @pl.when(i == 0)
    def _():
        out_ref[...] = jnp.zeros_like(out_ref)

    out_ref[...] += ps


def _tc_layer3_pool(m_p, nd, w, b):
    grid = NP // RB
    return pl.pallas_call(
        _pool_body,
        grid=(grid,),
        in_specs=[
            pl.BlockSpec((NC, RB, F), lambda i: (0, i, 0)),
            pl.BlockSpec((RB, 1), lambda i: (i, 0)),
            pl.BlockSpec((F, F), lambda i: (0, 0)),
            pl.BlockSpec((1, F), lambda i: (0, 0)),
        ],
        out_specs=pl.BlockSpec((1, F), lambda i: (0, 0)),
        out_shape=jax.ShapeDtypeStruct((1, F), jnp.float32),
    )(m_p, nd, w, b)


def _head_body(es_ref, fg_ref, w1a_ref, w1b_ref, b1_ref, w2_ref, b2_ref,
               w3_ref, b3_ref, out_ref):
    emb = es_ref[...] * (1.0 / N_NODES)
    t = (lax.dot_general(emb, w1a_ref[...], (((1,), (0,)), ((), ())),
                         precision=lax.Precision.HIGHEST,
                         preferred_element_type=jnp.float32)
         + lax.dot_general(fg_ref[...], w1b_ref[...], (((1,), (0,)), ((), ())),
                           precision=lax.Precision.HIGHEST,
                           preferred_element_type=jnp.float32))
    z = _selu(t + b1_ref[...])
    z = _selu(lax.dot_general(z, w2_ref[...], (((1,), (0,)), ((), ())),
                              precision=lax.Precision.HIGHEST,
                              preferred_element_type=jnp.float32)
              + b2_ref[...])
    out_ref[...] = (lax.dot_general(z, w3_ref[...], (((1,), (0,)), ((), ())),
                                    precision=lax.Precision.HIGHEST,
                                    preferred_element_type=jnp.float32)
                    + b3_ref[...])


def _tc_head(emb_sum, fg, w1a, w1b, b1, w2, b2, w3, b3):
    return pl.pallas_call(
        _head_body,
        out_shape=jax.ShapeDtypeStruct((1, 1), jnp.float32),
    )(emb_sum, fg, w1a, w1b, b1, w2, b2, w3, b3)


def kernel(feats_node, edge_index, feats_graph, W1, b1, W2, b2, W3, b3,
           Wm1, bm1, Wm2, bm2, Wm3, bm3):
    ei = edge_index.astype(jnp.int32)
    pad = jnp.full((EP - N_EDGES,), NP - 1, dtype=jnp.int32)
    src = jnp.concatenate([ei[0], pad])
    dst = jnp.concatenate([ei[1], pad])

    xp = jnp.zeros((NP, F), jnp.float32).at[:N_NODES].set(feats_node)

    dego, degi = _sc_degrees(src, dst)
    hs, ns, nd = _tc_norms(dego.T, degi.T, xp)

    m1 = _sc_aggregate(hs, src, dst)
    hs2 = _tc_layer(m1, nd, ns, W1, b1.reshape(1, F))

    m2 = _sc_aggregate(hs2, src, dst)
    hs3 = _tc_layer(m2, nd, ns, W2, b2.reshape(1, F))

    m3 = _sc_aggregate(hs3, src, dst)
    emb_sum = _tc_layer3_pool(m3, nd, W3, b3.reshape(1, F))

    return _tc_head(emb_sum, feats_graph,
                    Wm1[:F], Wm1[F:], bm1.reshape(1, 2 * F),
                    Wm2, bm2.reshape(1, F),
                    Wm3, bm3.reshape(1, 1))
